# Initial kernel scaffold; baseline (speedup 1.0000x reference)
import jax
import jax.numpy as jnp
from jax.experimental import pallas as pl


def kernel(feats, xyz, support_xyz, offset, support_offset, support_feats, ln1_g, ln1_b, W1, b1, ln2_g, ln2_b, W2, b2):
    raise NotImplementedError("write your pallas kernel here")


# speedup vs baseline: 12.1879x; 12.1879x over previous
import functools

import jax
import jax.numpy as jnp
from jax import lax
from jax.experimental import pallas as pl
from jax.experimental.pallas import tpu as pltpu
from jax.experimental.pallas import tpu_sc as plsc

NC_PTS = 4096
NF_PTS = 16384
CIN = 512
COUT = 256
KNN = 3

_HIGH = jax.lax.Precision.HIGHEST


def _h2_body(feats_ref, g_ref, b_ref, w2_ref, b2_ref, out_ref):
    x = feats_ref[...]
    m = jnp.mean(x, axis=-1, keepdims=True)
    v = jnp.mean((x - m) * (x - m), axis=-1, keepdims=True)
    xn = (x - m) / jnp.sqrt(v + 1e-5) * g_ref[...] + b_ref[...]
    out_ref[...] = (
        jnp.dot(xn, w2_ref[...], preferred_element_type=jnp.float32,
                precision=_HIGH)
        + b2_ref[...]
    )


def _knn_body(sxyz_ref, xyzt_ref, idx_ref, w_ref):
    s = sxyz_ref[...]
    x = xyzt_ref[...]
    br = s.shape[0]
    s2 = jnp.sum(s * s, axis=1, keepdims=True)
    x2 = jnp.sum(x * x, axis=0, keepdims=True)
    sb = s.astype(jnp.bfloat16).astype(jnp.float32)
    xb = x.astype(jnp.bfloat16).astype(jnp.float32)
    dot = (sb[:, 0:1] * xb[0:1, :]
           + sb[:, 1:2] * xb[1:2, :]
           + sb[:, 2:3] * xb[2:3, :])
    d = s2 + x2 - 2.0 * dot

    cols = lax.broadcasted_iota(jnp.int32, (br, NC_PTS), 1)
    vals = []
    idxs = []
    for _ in range(KNN):
        m = jnp.min(d, axis=1, keepdims=True)
        i = jnp.min(jnp.where(d == m, cols, NC_PTS), axis=1,
                    keepdims=True)
        vals.append(m)
        idxs.append(i)
        d = jnp.where(cols == i, jnp.inf, d)

    dist = [jnp.sqrt(jnp.maximum(v, 0.0)) for v in vals]
    u = [1.0 / (dk + 1e-8) for dk in dist]
    usum = u[0] + u[1] + u[2]
    w = [uk / usum for uk in u]

    idx_ref[...] = jnp.concatenate(idxs, axis=1)
    w_ref[...] = jnp.concatenate(w, axis=1)


_GATHER_WINDOW = 128


def _sc_gather(h2, idx_flat):
    num_idx = idx_flat.shape[1]
    mesh = plsc.VectorSubcoreMesh(core_axis_name="core",
                                  subcore_axis_name="subcore")

    @functools.partial(
        pl.kernel,
        out_type=jax.ShapeDtypeStruct((num_idx, COUT), jnp.float32),
        mesh=mesh,
    )
    def gather_kernel(h2_hbm, i_hbm, o_hbm):
        def body(i_vmem, o_vmem):
            pltpu.sync_copy(h2_hbm.at[i_vmem.at[0]], o_vmem)

        pltpu.emit_pipeline(
            body,
            grid=(num_idx // _GATHER_WINDOW,),
            in_specs=[pl.BlockSpec((1, _GATHER_WINDOW),
                                   index_map=lambda i: (0, i))],
            out_specs=[pl.BlockSpec((_GATHER_WINDOW, COUT),
                                    index_map=lambda i: (i, 0))],
            core_axis_name=("core", "subcore"),
            dimension_semantics=(pltpu.PARALLEL,),
        )(i_hbm, o_hbm)

    return gather_kernel(h2, idx_flat)


def _final_body(sf_ref, g_ref, b_ref, w1_ref, b1_ref, w_ref,
                g0_ref, g1_ref, g2_ref, out_ref):
    x = sf_ref[...]
    m = jnp.mean(x, axis=-1, keepdims=True)
    v = jnp.mean((x - m) * (x - m), axis=-1, keepdims=True)
    xn = (x - m) / jnp.sqrt(v + 1e-5) * g_ref[...] + b_ref[...]
    h1 = (jnp.dot(xn, w1_ref[...], preferred_element_type=jnp.float32,
                  precision=_HIGH)
          + b1_ref[...])
    interp = (w_ref[:, 0:1] * g0_ref[...]
              + w_ref[:, 1:2] * g1_ref[...]
              + w_ref[:, 2:3] * g2_ref[...])
    out_ref[...] = h1 + interp


def kernel(feats, xyz, support_xyz, offset, support_offset, support_feats,
           ln1_g, ln1_b, W1, b1, ln2_g, ln2_b, W2, b2):
    h2 = pl.pallas_call(
        _h2_body,
        out_shape=jax.ShapeDtypeStruct((NC_PTS, COUT), jnp.float32),
    )(feats, ln2_g, ln2_b, W2, b2)

    BR = 512
    xyzt = xyz.T
    idx3, w3 = pl.pallas_call(
        _knn_body,
        grid=(NF_PTS // BR,),
        in_specs=[
            pl.BlockSpec((BR, 3), lambda i: (i, 0)),
            pl.BlockSpec((3, NC_PTS), lambda i: (0, 0)),
        ],
        out_specs=[
            pl.BlockSpec((BR, KNN), lambda i: (i, 0)),
            pl.BlockSpec((BR, KNN), lambda i: (i, 0)),
        ],
        out_shape=[
            jax.ShapeDtypeStruct((NF_PTS, KNN), jnp.int32),
            jax.ShapeDtypeStruct((NF_PTS, KNN), jnp.float32),
        ],
    )(support_xyz, xyzt)

    idx_flat = idx3.T.reshape(1, KNN * NF_PTS)
    gathered = _sc_gather(h2, idx_flat)

    BF = 2048
    nsteps = NF_PTS // BF
    out = pl.pallas_call(
        _final_body,
        grid=(nsteps,),
        in_specs=[
            pl.BlockSpec((BF, COUT), lambda i: (i, 0)),
            pl.BlockSpec((COUT,), lambda i: (0,)),
            pl.BlockSpec((COUT,), lambda i: (0,)),
            pl.BlockSpec((COUT, COUT), lambda i: (0, 0)),
            pl.BlockSpec((COUT,), lambda i: (0,)),
            pl.BlockSpec((BF, KNN), lambda i: (i, 0)),
            pl.BlockSpec((BF, COUT), lambda i: (i, 0)),
            pl.BlockSpec((BF, COUT), lambda i: (nsteps + i, 0)),
            pl.BlockSpec((BF, COUT), lambda i: (2 * nsteps + i, 0)),
        ],
        out_specs=pl.BlockSpec((BF, COUT), lambda i: (i, 0)),
        out_shape=jax.ShapeDtypeStruct((NF_PTS, COUT), jnp.float32),
    )(support_feats, ln1_g, ln1_b, W1, b1, w3, gathered, gathered, gathered)

    return (out, support_xyz, support_offset)
